# Initial kernel scaffold; baseline (speedup 1.0000x reference)
import jax
import jax.numpy as jnp
from jax.experimental import pallas as pl


def kernel(inputs, proj_w, proj_b, head_w, head_b, cls_w, cls_b, bbox_w, bbox_b, ctr_w, ctr_b, scales):
    raise NotImplementedError("write your pallas kernel here")


# speedup vs baseline: 1.7673x; 1.7673x over previous
import functools

import jax
import jax.numpy as jnp
import numpy as np
from jax import lax
from jax.experimental import pallas as pl
from jax.experimental.pallas import tpu as pltpu

BT = 16
C = 256
P0, P1S, P2S = 256, 64, 16


def _pool_matrix(side):
    half = side // 2
    m = np.zeros((half * half, side * side), dtype=np.float32)
    for r in range(half):
        for c in range(half):
            for i in range(2):
                for j in range(2):
                    m[r * half + c, (2 * r + i) * side + (2 * c + j)] = 0.25
    return m


def _conv_head(xl, side, lvl, hw_ref, hb_ref, w7_ref, b7_ref):
    n = xl.shape[0]
    pp = side * side
    p = lax.broadcasted_iota(jnp.int32, (n, 1), 0) % pp
    r = p // side
    c = p % side
    acc = jnp.zeros((n, C), jnp.float32)
    for dr in (-1, 0, 1):
        for dc in (-1, 0, 1):
            d = (dr + 1) * 3 + (dc + 1)
            delta = dr * side + dc
            y = xl if delta == 0 else jnp.roll(xl, -delta, axis=0)
            conds = []
            if dr == -1:
                conds.append(r >= 1)
            elif dr == 1:
                conds.append(r <= side - 2)
            if dc == -1:
                conds.append(c >= 1)
            elif dc == 1:
                conds.append(c <= side - 2)
            if conds:
                m = conds[0]
                for extra in conds[1:]:
                    m = m & extra
                y = jnp.where(m, y, 0.0)
            acc = acc + jnp.dot(y, hw_ref[d], preferred_element_type=jnp.float32)
    h = jnp.maximum(acc + hb_ref[...], 0.0)
    out = jnp.dot(h, w7_ref[lvl], preferred_element_type=jnp.float32) + b7_ref[lvl]
    ci = lax.broadcasted_iota(jnp.int32, (1, 7), 1)
    is_bbox = (ci >= 2) & (ci < 6)
    return jnp.where(is_bbox, jnp.exp(out), out)


def _detector_body(in_ref, pw_ref, pb_ref, hw_ref, hb_ref, w7_ref, b7_ref,
                   pool1_ref, pool2_ref, out_ref):
    x = in_ref[...].reshape(BT * P0, in_ref.shape[2])
    x0 = jnp.dot(x, pw_ref[...], preferred_element_type=jnp.float32) + pb_ref[...]

    out0 = _conv_head(x0, 16, 0, hw_ref, hb_ref, w7_ref, b7_ref)

    pool1 = pool1_ref[...]
    x1 = jnp.concatenate(
        [jnp.dot(pool1, x0[i * P0:(i + 1) * P0], preferred_element_type=jnp.float32)
         for i in range(BT)], axis=0)
    out1 = _conv_head(x1, 8, 1, hw_ref, hb_ref, w7_ref, b7_ref)

    pool2 = pool2_ref[...]
    x2 = jnp.concatenate(
        [jnp.dot(pool2, x1[i * P1S:(i + 1) * P1S], preferred_element_type=jnp.float32)
         for i in range(BT)], axis=0)
    out2 = _conv_head(x2, 4, 2, hw_ref, hb_ref, w7_ref, b7_ref)

    out_ref[:, 0:P0, :] = out0.reshape(BT, P0, 7)
    out_ref[:, P0:P0 + P1S, :] = out1.reshape(BT, P1S, 7)
    out_ref[:, P0 + P1S:P0 + P1S + P2S, :] = out2.reshape(BT, P2S, 7)


@jax.jit
def kernel(inputs, proj_w, proj_b, head_w, head_b, cls_w, cls_b,
           bbox_w, bbox_b, ctr_w, ctr_b, scales):
    B, T, D = inputs.shape

    hw = head_w.reshape(9, C, C)
    pb = proj_b.reshape(1, C)
    hb = head_b.reshape(1, C)
    w_cls = cls_w.reshape(C, 2)
    w_bbox = bbox_w.reshape(C, 4)
    w_ctr = ctr_w.reshape(C, 1)
    w7 = jnp.stack([
        jnp.concatenate([w_cls, w_bbox * scales[l], w_ctr], axis=1)
        for l in range(3)], axis=0)
    b7 = jnp.stack([
        jnp.concatenate([cls_b, bbox_b * scales[l], ctr_b], axis=0).reshape(1, 7)
        for l in range(3)], axis=0)
    pool1 = jnp.asarray(_pool_matrix(16))
    pool2 = jnp.asarray(_pool_matrix(8))

    grid = (B // BT,)
    out = pl.pallas_call(
        _detector_body,
        grid=grid,
        in_specs=[
            pl.BlockSpec((BT, T, D), lambda i: (i, 0, 0)),
            pl.BlockSpec((D, C), lambda i: (0, 0)),
            pl.BlockSpec((1, C), lambda i: (0, 0)),
            pl.BlockSpec((9, C, C), lambda i: (0, 0, 0)),
            pl.BlockSpec((1, C), lambda i: (0, 0)),
            pl.BlockSpec((3, C, 7), lambda i: (0, 0, 0)),
            pl.BlockSpec((3, 1, 7), lambda i: (0, 0, 0)),
            pl.BlockSpec((P1S, P0), lambda i: (0, 0)),
            pl.BlockSpec((P2S, P1S), lambda i: (0, 0)),
        ],
        out_specs=pl.BlockSpec((BT, 336, 7), lambda i: (i, 0, 0)),
        out_shape=jax.ShapeDtypeStruct((B, 336, 7), jnp.float32),
        compiler_params=pltpu.CompilerParams(
            dimension_semantics=("arbitrary",),
        ),
    )(inputs, proj_w, pb, hw, hb, w7, b7, pool1, pool2)

    return out.transpose(0, 2, 1)
